# Initial kernel scaffold; baseline (speedup 1.0000x reference)
import jax
import jax.numpy as jnp
from jax.experimental import pallas as pl


def kernel(x, embedding_matrix):
    raise NotImplementedError("write your pallas kernel here")


# speedup vs baseline: 1.4794x; 1.4794x over previous
import functools

import jax
import jax.numpy as jnp
from jax import lax
from jax.experimental import pallas as pl
from jax.experimental.pallas import tpu as pltpu
from jax.experimental.pallas import tpu_sc as plsc


def _make_emb_kernel(N, D, n_per_w, chunk, num_cores):
    n_ch = n_per_w // chunk
    mesh = plsc.VectorSubcoreMesh(core_axis_name="c", subcore_axis_name="s")

    @functools.partial(
        pl.kernel,
        mesh=mesh,
        out_type=jax.ShapeDtypeStruct((N, D), jnp.float32),
        scratch_types=[
            pltpu.VMEM((chunk,), jnp.int32),
            pltpu.VMEM((chunk, D), jnp.float32),
            pltpu.SemaphoreType.DMA,
        ],
        compiler_params=pltpu.CompilerParams(use_tc_tiling_on_sc=False),
    )
    def emb(idx_hbm, table_hbm, out_hbm, idx_v, rows_v, sem):
        wid = lax.axis_index("s") * num_cores + lax.axis_index("c")
        base = wid * n_per_w

        def body(ci, carry):
            off = base + ci * chunk
            pltpu.sync_copy(idx_hbm.at[pl.ds(off, chunk)], idx_v)
            pltpu.async_copy(table_hbm.at[idx_v], rows_v, sem).wait()
            pltpu.sync_copy(rows_v, out_hbm.at[pl.ds(off, chunk)])
            return carry

        lax.fori_loop(0, n_ch, body, 0)

    return emb


def kernel(x, embedding_matrix):
    B, H = x.shape
    V, D = embedding_matrix.shape
    N = B * H
    idx = x.reshape(N).astype(jnp.int32)

    info = plsc.get_sparse_core_info()
    nw = info.num_cores * info.num_subcores
    n_per_w = N // nw
    chunk = 1600

    emb = _make_emb_kernel(N, D, n_per_w, chunk, info.num_cores)
    out = emb(idx, embedding_matrix)
    return out.reshape(B, H, D)


# speedup vs baseline: 1.4948x; 1.0104x over previous
import functools

import jax
import jax.numpy as jnp
from jax import lax
from jax.experimental import pallas as pl
from jax.experimental.pallas import tpu as pltpu
from jax.experimental.pallas import tpu_sc as plsc


def _make_emb_kernel(N, D, n_per_w, chunk, num_cores):
    n_ch = n_per_w // chunk
    assert n_ch % 2 == 0 and n_ch * chunk == n_per_w
    mesh = plsc.VectorSubcoreMesh(core_axis_name="c", subcore_axis_name="s")

    @functools.partial(
        pl.kernel,
        mesh=mesh,
        out_type=jax.ShapeDtypeStruct((N, D), jnp.float32),
        scratch_types=[
            pltpu.VMEM((n_per_w,), jnp.int32),
            pltpu.VMEM((chunk, D), jnp.float32),
            pltpu.VMEM((chunk, D), jnp.float32),
            pltpu.SemaphoreType.DMA,
            pltpu.SemaphoreType.DMA,
            pltpu.SemaphoreType.DMA,
            pltpu.SemaphoreType.DMA,
        ],
        compiler_params=pltpu.CompilerParams(use_tc_tiling_on_sc=False),
    )
    def emb(idx_hbm, table_hbm, out_hbm, idx_v, r0, r1, g0, g1, o0, o1):
        wid = lax.axis_index("s") * num_cores + lax.axis_index("c")
        base = wid * n_per_w

        pltpu.sync_copy(idx_hbm.at[pl.ds(base, n_per_w)], idx_v)

        def gather(c, buf, sem):
            return pltpu.make_async_copy(
                table_hbm.at[idx_v.at[pl.ds(c * chunk, chunk)]], buf, sem
            )

        def outcp(c, buf, sem):
            return pltpu.make_async_copy(
                buf, out_hbm.at[pl.ds(base + c * chunk, chunk)], sem
            )

        gather(0, r0, g0).start()

        def body(i, carry):
            c0 = 2 * i
            c1 = 2 * i + 1
            gather(c0, r0, g0).wait()

            @pl.when(i >= 1)
            def _():
                outcp(c1 - 2, r1, o1).wait()

            gather(c1, r1, g1).start()
            outcp(c0, r0, o0).start()

            gather(c1, r1, g1).wait()
            outcp(c0, r0, o0).wait()

            @pl.when(i < (n_ch // 2 - 1))
            def _():
                gather(c0 + 2, r0, g0).start()

            outcp(c1, r1, o1).start()
            return carry

        lax.fori_loop(0, n_ch // 2, body, 0)
        outcp(n_ch - 1, r1, o1).wait()

    return emb


def kernel(x, embedding_matrix):
    B, H = x.shape
    V, D = embedding_matrix.shape
    N = B * H
    idx = x.reshape(N).astype(jnp.int32)

    info = plsc.get_sparse_core_info()
    nw = info.num_cores * info.num_subcores
    n_per_w = N // nw
    chunk = 1600

    emb = _make_emb_kernel(N, D, n_per_w, chunk, info.num_cores)
    out = emb(idx, embedding_matrix)
    return out.reshape(B, H, D)
